# Initial kernel scaffold; baseline (speedup 1.0000x reference)
import jax
import jax.numpy as jnp
from jax.experimental import pallas as pl


def kernel(x, adj, emb, W1_1, W2_1, b1, W1_h, W2_h, bh, W1_2, W2_2, b2):
    raise NotImplementedError("write your pallas kernel here")


# speedup vs baseline: 1.0380x; 1.0380x over previous
import jax
import jax.numpy as jnp
from jax.experimental import pallas as pl
from jax.experimental.pallas import tpu as pltpu


def _hidden_body(adj_ref, emb_hbm_ref, embrow_ref, w1_ref, w2_ref, b_ref,
                 h2f_ref, h2bf_ref, emb_vmem, sem):
    @pl.when(pl.program_id(0) == 0)
    def _load():
        cp = pltpu.make_async_copy(emb_hbm_ref, emb_vmem, sem)
        cp.start()
        cp.wait()

    ax = jnp.dot(adj_ref[...].astype(jnp.bfloat16), emb_vmem[...],
                 preferred_element_type=jnp.float32)
    t = jnp.dot(ax.astype(jnp.bfloat16), w1_ref[...],
                preferred_element_type=jnp.float32)
    t += jnp.dot((ax * embrow_ref[...]).astype(jnp.bfloat16), w2_ref[...],
                 preferred_element_type=jnp.float32)
    h = jnp.maximum(t + b_ref[...], 0.0)
    h2f_ref[...] = h
    h2bf_ref[...] = h.astype(jnp.bfloat16)


def _out_body(adj_ref, h2_hbm_ref, h2row_ref, w1_ref, w2_ref, b_ref,
              out_ref, h2_vmem, sem):
    @pl.when(pl.program_id(0) == 0)
    def _load():
        cp = pltpu.make_async_copy(h2_hbm_ref, h2_vmem, sem)
        cp.start()
        cp.wait()

    ax = jnp.dot(adj_ref[...].astype(jnp.bfloat16), h2_vmem[...],
                 preferred_element_type=jnp.float32)
    t = jnp.dot(ax, w1_ref[...], preferred_element_type=jnp.float32,
                precision=jax.lax.Precision.HIGHEST)
    t += jnp.dot(ax * h2row_ref[...], w2_ref[...],
                 preferred_element_type=jnp.float32,
                 precision=jax.lax.Precision.HIGHEST)
    o = t + b_ref[...]
    m = jnp.max(o, axis=1, keepdims=True)
    e = o - m
    lse = jnp.log(jnp.sum(jnp.exp(e), axis=1, keepdims=True))
    out_ref[...] = e - lse


def _forward(adj, emb, w1h, w2h, bh, w12, w22, b2, bm):
    n, d = emb.shape
    h = w1h.shape[1]
    c = w12.shape[1]
    assert n % bm == 0
    grid = (n // bm,)

    emb_bf = emb.astype(jnp.bfloat16)
    h2f, h2bf = pl.pallas_call(
        _hidden_body,
        grid=grid,
        in_specs=[
            pl.BlockSpec((bm, n), lambda i: (i, 0)),
            pl.BlockSpec(memory_space=pl.ANY),
            pl.BlockSpec((bm, d), lambda i: (i, 0)),
            pl.BlockSpec((d, h), lambda i: (0, 0)),
            pl.BlockSpec((d, h), lambda i: (0, 0)),
            pl.BlockSpec((1, h), lambda i: (0, 0)),
        ],
        out_specs=[
            pl.BlockSpec((bm, h), lambda i: (i, 0)),
            pl.BlockSpec((bm, h), lambda i: (i, 0)),
        ],
        out_shape=[
            jax.ShapeDtypeStruct((n, h), jnp.float32),
            jax.ShapeDtypeStruct((n, h), jnp.bfloat16),
        ],
        scratch_shapes=[pltpu.VMEM((n, d), jnp.bfloat16),
                        pltpu.SemaphoreType.DMA],
    )(adj, emb_bf, emb, w1h.astype(jnp.bfloat16), w2h.astype(jnp.bfloat16),
      bh.reshape(1, h))

    out = pl.pallas_call(
        _out_body,
        grid=grid,
        in_specs=[
            pl.BlockSpec((bm, n), lambda i: (i, 0)),
            pl.BlockSpec(memory_space=pl.ANY),
            pl.BlockSpec((bm, h), lambda i: (i, 0)),
            pl.BlockSpec((h, c), lambda i: (0, 0)),
            pl.BlockSpec((h, c), lambda i: (0, 0)),
            pl.BlockSpec((1, c), lambda i: (0, 0)),
        ],
        out_specs=pl.BlockSpec((bm, c), lambda i: (i, 0)),
        out_shape=jax.ShapeDtypeStruct((n, c), jnp.float32),
        scratch_shapes=[pltpu.VMEM((n, h), jnp.bfloat16),
                        pltpu.SemaphoreType.DMA],
    )(adj, h2bf, h2f, w12, w22, b2.reshape(1, c))
    return out


def kernel(x, adj, emb, W1_1, W2_1, b1, W1_h, W2_h, bh, W1_2, W2_2, b2):
    del x, W1_1, W2_1, b1
    return _forward(adj, emb, W1_h, W2_h, bh, W1_2, W2_2, b2, bm=200)


# speedup vs baseline: 1.0466x; 1.0084x over previous
import functools

import jax
import jax.numpy as jnp
from jax.experimental import pallas as pl
from jax.experimental.pallas import tpu as pltpu


def _hidden_body(adj_ref, emb_hbm_ref, w1_hbm_ref, w2_hbm_ref, b_ref,
                 h2bf_ref, emb_vmem, w1_vmem, w2_vmem, sems, *, bm):
    i = pl.program_id(0)

    @pl.when(i == 0)
    def _load():
        cps = [pltpu.make_async_copy(emb_hbm_ref, emb_vmem, sems.at[0]),
               pltpu.make_async_copy(w1_hbm_ref, w1_vmem, sems.at[1]),
               pltpu.make_async_copy(w2_hbm_ref, w2_vmem, sems.at[2])]
        for cp in cps:
            cp.start()
        for cp in cps:
            cp.wait()

    ax = jnp.dot(adj_ref[...].astype(jnp.bfloat16), emb_vmem[...],
                 preferred_element_type=jnp.float32)
    t = jnp.dot(ax.astype(jnp.bfloat16), w1_vmem[...],
                preferred_element_type=jnp.float32)
    rows = emb_vmem[pl.ds(i * bm, bm), :].astype(jnp.float32)
    t += jnp.dot((ax * rows).astype(jnp.bfloat16), w2_vmem[...],
                 preferred_element_type=jnp.float32)
    h = jnp.maximum(t + b_ref[...], 0.0)
    h2bf_ref[...] = h.astype(jnp.bfloat16)


def _out_body(adj_ref, h2_hbm_ref, w1_ref, w2_ref, b_ref,
              out_ref, h2_vmem, sem, *, bm):
    i = pl.program_id(0)

    @pl.when(i == 0)
    def _load():
        cp = pltpu.make_async_copy(h2_hbm_ref, h2_vmem, sem)
        cp.start()
        cp.wait()

    ax = jnp.dot(adj_ref[...].astype(jnp.bfloat16), h2_vmem[...],
                 preferred_element_type=jnp.float32)
    t = jnp.dot(ax, w1_ref[...], preferred_element_type=jnp.float32,
                precision=jax.lax.Precision.HIGHEST)
    rows = h2_vmem[pl.ds(i * bm, bm), :].astype(jnp.float32)
    t += jnp.dot(ax * rows, w2_ref[...],
                 preferred_element_type=jnp.float32,
                 precision=jax.lax.Precision.HIGHEST)
    o = t + b_ref[...]
    m = jnp.max(o, axis=1, keepdims=True)
    e = o - m
    lse = jnp.log(jnp.sum(jnp.exp(e), axis=1, keepdims=True))
    out_ref[...] = e - lse


def _forward(adj, emb, w1h, w2h, bh, w12, w22, b2, bm):
    n, d = emb.shape
    h = w1h.shape[1]
    c = w12.shape[1]
    assert n % bm == 0
    grid = (n // bm,)

    emb_bf = emb.astype(jnp.bfloat16)
    h2bf = pl.pallas_call(
        functools.partial(_hidden_body, bm=bm),
        grid=grid,
        in_specs=[
            pl.BlockSpec((bm, n), lambda i: (i, 0)),
            pl.BlockSpec(memory_space=pl.ANY),
            pl.BlockSpec(memory_space=pl.ANY),
            pl.BlockSpec(memory_space=pl.ANY),
            pl.BlockSpec((1, h), lambda i: (0, 0)),
        ],
        out_specs=pl.BlockSpec((bm, h), lambda i: (i, 0)),
        out_shape=jax.ShapeDtypeStruct((n, h), jnp.bfloat16),
        scratch_shapes=[pltpu.VMEM((n, d), jnp.bfloat16),
                        pltpu.VMEM((d, h), jnp.bfloat16),
                        pltpu.VMEM((d, h), jnp.bfloat16),
                        pltpu.SemaphoreType.DMA((3,))],
    )(adj, emb_bf, w1h.astype(jnp.bfloat16), w2h.astype(jnp.bfloat16),
      bh.reshape(1, h))

    out = pl.pallas_call(
        functools.partial(_out_body, bm=bm),
        grid=grid,
        in_specs=[
            pl.BlockSpec((bm, n), lambda i: (i, 0)),
            pl.BlockSpec(memory_space=pl.ANY),
            pl.BlockSpec((h, c), lambda i: (0, 0)),
            pl.BlockSpec((h, c), lambda i: (0, 0)),
            pl.BlockSpec((1, c), lambda i: (0, 0)),
        ],
        out_specs=pl.BlockSpec((bm, c), lambda i: (i, 0)),
        out_shape=jax.ShapeDtypeStruct((n, c), jnp.float32),
        scratch_shapes=[pltpu.VMEM((n, h), jnp.bfloat16),
                        pltpu.SemaphoreType.DMA],
    )(adj, h2bf, w12, w22, b2.reshape(1, c))
    return out


def kernel(x, adj, emb, W1_1, W2_1, b1, W1_h, W2_h, bh, W1_2, W2_2, b2):
    del x, W1_1, W2_1, b1
    return _forward(adj, emb, W1_h, W2_h, bh, W1_2, W2_2, b2, bm=200)


# speedup vs baseline: 1.1878x; 1.1349x over previous
import functools

import jax
import jax.numpy as jnp
from jax.experimental import pallas as pl
from jax.experimental.pallas import tpu as pltpu


def _hidden_body(adj_ref, emb_hbm_ref, w1_hbm_ref, w2_hbm_ref, b_ref,
                 h2bf_ref, adjbf_ref, emb_vmem, w1_vmem, w2_vmem, sems, *, bm):
    i = pl.program_id(0)

    @pl.when(i == 0)
    def _load():
        cps = [pltpu.make_async_copy(emb_hbm_ref, emb_vmem, sems.at[0]),
               pltpu.make_async_copy(w1_hbm_ref, w1_vmem, sems.at[1]),
               pltpu.make_async_copy(w2_hbm_ref, w2_vmem, sems.at[2])]
        for cp in cps:
            cp.start()
        for cp in cps:
            cp.wait()

    adjb = adj_ref[...].astype(jnp.bfloat16)
    adjbf_ref[...] = adjb
    ax = jnp.dot(adjb, emb_vmem[...], preferred_element_type=jnp.float32)
    axb = ax.astype(jnp.bfloat16)
    t = jnp.dot(axb, w1_vmem[...], preferred_element_type=jnp.float32)
    rows = emb_vmem[pl.ds(pl.multiple_of(i * bm, 16), bm), :]
    t += jnp.dot(axb * rows, w2_vmem[...],
                 preferred_element_type=jnp.float32)
    h = jnp.maximum(t + b_ref[...], 0.0)
    h2bf_ref[...] = h.astype(jnp.bfloat16)


def _out_body(adj_ref, h2_hbm_ref, w1_ref, w2_ref, b_ref,
              out_ref, h2_vmem, sem, *, bm):
    i = pl.program_id(0)

    @pl.when(i == 0)
    def _load():
        cp = pltpu.make_async_copy(h2_hbm_ref, h2_vmem, sem)
        cp.start()
        cp.wait()

    ax = jnp.dot(adj_ref[...], h2_vmem[...],
                 preferred_element_type=jnp.float32)
    axb = ax.astype(jnp.bfloat16)
    t = jnp.dot(axb, w1_ref[...], preferred_element_type=jnp.float32)
    rows = h2_vmem[pl.ds(pl.multiple_of(i * bm, 16), bm), :]
    t += jnp.dot(axb * rows, w2_ref[...], preferred_element_type=jnp.float32)
    o = t + b_ref[...]
    m = jnp.max(o, axis=1, keepdims=True)
    e = o - m
    lse = jnp.log(jnp.sum(jnp.exp(e), axis=1, keepdims=True))
    out_ref[...] = e - lse


def _forward(adj, emb, w1h, w2h, bh, w12, w22, b2, bm1, bm2):
    n, d = emb.shape
    h = w1h.shape[1]
    c = w12.shape[1]

    assert n % bm1 == 0 and n % bm2 == 0
    bm = bm1
    emb_bf = emb.astype(jnp.bfloat16)
    h2bf, adj_bf = pl.pallas_call(
        functools.partial(_hidden_body, bm=bm),
        grid=(n // bm,),
        in_specs=[
            pl.BlockSpec((bm, n), lambda i: (i, 0)),
            pl.BlockSpec(memory_space=pl.ANY),
            pl.BlockSpec(memory_space=pl.ANY),
            pl.BlockSpec(memory_space=pl.ANY),
            pl.BlockSpec((1, h), lambda i: (0, 0)),
        ],
        out_specs=[
            pl.BlockSpec((bm, h), lambda i: (i, 0)),
            pl.BlockSpec((bm, n), lambda i: (i, 0)),
        ],
        out_shape=[
            jax.ShapeDtypeStruct((n, h), jnp.bfloat16),
            jax.ShapeDtypeStruct((n, n), jnp.bfloat16),
        ],
        scratch_shapes=[pltpu.VMEM((n, d), jnp.bfloat16),
                        pltpu.VMEM((d, h), jnp.bfloat16),
                        pltpu.VMEM((d, h), jnp.bfloat16),
                        pltpu.SemaphoreType.DMA((3,))],
    )(adj, emb_bf, w1h.astype(jnp.bfloat16), w2h.astype(jnp.bfloat16),
      bh.reshape(1, h))

    bm = bm2
    out = pl.pallas_call(
        functools.partial(_out_body, bm=bm),
        grid=(n // bm,),
        in_specs=[
            pl.BlockSpec((bm, n), lambda i: (i, 0)),
            pl.BlockSpec(memory_space=pl.ANY),
            pl.BlockSpec((h, c), lambda i: (0, 0)),
            pl.BlockSpec((h, c), lambda i: (0, 0)),
            pl.BlockSpec((1, c), lambda i: (0, 0)),
        ],
        out_specs=pl.BlockSpec((bm, c), lambda i: (i, 0)),
        out_shape=jax.ShapeDtypeStruct((n, c), jnp.float32),
        scratch_shapes=[pltpu.VMEM((n, h), jnp.bfloat16),
                        pltpu.SemaphoreType.DMA],
    )(adj_bf, h2bf, w12.astype(jnp.bfloat16), w22.astype(jnp.bfloat16),
      b2.reshape(1, c))
    return out


def kernel(x, adj, emb, W1_1, W2_1, b1, W1_h, W2_h, bh, W1_2, W2_2, b2):
    del x, W1_1, W2_1, b1
    return _forward(adj, emb, W1_h, W2_h, bh, W1_2, W2_2, b2, bm1=200, bm2=400)


# speedup vs baseline: 1.2041x; 1.0137x over previous
import functools

import jax
import jax.numpy as jnp
from jax.experimental import pallas as pl
from jax.experimental.pallas import tpu as pltpu


_CHUNK = 400


def _hidden_body(adj_ref, emb_hbm_ref, wcat_hbm_ref, b_ref,
                 h2bf_ref, adjbf_ref, emb_vmem, wcat_vmem, stage, sems,
                 *, bm):
    i = pl.program_id(0)

    @pl.when(i == 0)
    def _load():
        nchunks = emb_hbm_ref.shape[0] // _CHUNK

        def _chunk_copy(c, buf):
            return pltpu.make_async_copy(
                emb_hbm_ref.at[pl.ds(pl.multiple_of(c * _CHUNK, 16), _CHUNK)],
                stage.at[buf], sems.at[buf])

        wcp = pltpu.make_async_copy(wcat_hbm_ref, wcat_vmem, sems.at[2])
        wcp.start()
        _chunk_copy(0, 0).start()
        _chunk_copy(1, 1).start()

        def _body(c, carry):
            buf = jax.lax.rem(c, 2)
            _chunk_copy(c, buf).wait()
            emb_vmem[pl.ds(pl.multiple_of(c * _CHUNK, 16), _CHUNK), :] = (
                stage[buf].astype(jnp.bfloat16))

            @pl.when(c + 2 < nchunks)
            def _next():
                _chunk_copy(c + 2, buf).start()

            return carry

        jax.lax.fori_loop(0, nchunks, _body, 0)
        wcp.wait()

    adjb = adj_ref[...].astype(jnp.bfloat16)
    adjbf_ref[...] = adjb
    ax = jnp.dot(adjb, emb_vmem[...], preferred_element_type=jnp.float32)
    axb = ax.astype(jnp.bfloat16)
    rows = emb_vmem[pl.ds(pl.multiple_of(i * bm, 16), bm), :]
    lhs = jnp.concatenate([axb, axb * rows], axis=1)
    t = jnp.dot(lhs, wcat_vmem[...], preferred_element_type=jnp.float32)
    h = jnp.maximum(t + b_ref[...], 0.0)
    h2bf_ref[...] = h.astype(jnp.bfloat16)


def _out_body(adj_ref, h2_hbm_ref, wcat_ref, b_ref,
              out_ref, h2_vmem, sem, *, bm):
    i = pl.program_id(0)

    @pl.when(i == 0)
    def _load():
        cp = pltpu.make_async_copy(h2_hbm_ref, h2_vmem, sem)
        cp.start()
        cp.wait()

    ax = jnp.dot(adj_ref[...], h2_vmem[...],
                 preferred_element_type=jnp.float32)
    axb = ax.astype(jnp.bfloat16)
    rows = h2_vmem[pl.ds(pl.multiple_of(i * bm, 16), bm), :]
    hdim = rows.shape[1]
    o = jnp.dot(axb, wcat_ref[0:hdim, :], preferred_element_type=jnp.float32)
    o += jnp.dot(axb * rows, wcat_ref[hdim:2 * hdim, :],
                 preferred_element_type=jnp.float32)
    o = o + b_ref[...]
    m = jnp.max(o, axis=1, keepdims=True)
    e = o - m
    lse = jnp.log(jnp.sum(jnp.exp(e), axis=1, keepdims=True))
    out_ref[...] = e - lse


def _forward(adj, emb, w1h, w2h, bh, w12, w22, b2, bm1, bm2):
    n, d = emb.shape
    h = w1h.shape[1]
    c = w12.shape[1]

    assert n % bm1 == 0 and n % bm2 == 0 and n % _CHUNK == 0
    bm = bm1
    h2bf, adj_bf = pl.pallas_call(
        functools.partial(_hidden_body, bm=bm),
        grid=(n // bm,),
        in_specs=[
            pl.BlockSpec((bm, n), lambda i: (i, 0)),
            pl.BlockSpec(memory_space=pl.ANY),
            pl.BlockSpec(memory_space=pl.ANY),
            pl.BlockSpec((1, h), lambda i: (0, 0)),
        ],
        out_specs=[
            pl.BlockSpec((bm, h), lambda i: (i, 0)),
            pl.BlockSpec((bm, n), lambda i: (i, 0)),
        ],
        out_shape=[
            jax.ShapeDtypeStruct((n, h), jnp.bfloat16),
            jax.ShapeDtypeStruct((n, n), jnp.bfloat16),
        ],
        scratch_shapes=[pltpu.VMEM((n, d), jnp.bfloat16),
                        pltpu.VMEM((2 * d, h), jnp.bfloat16),
                        pltpu.VMEM((2, _CHUNK, d), jnp.float32),
                        pltpu.SemaphoreType.DMA((3,))],
    )(adj, emb,
      jnp.concatenate([w1h, w2h], axis=0).astype(jnp.bfloat16),
      bh.reshape(1, h))

    bm = bm2
    out = pl.pallas_call(
        functools.partial(_out_body, bm=bm),
        grid=(n // bm,),
        in_specs=[
            pl.BlockSpec((bm, n), lambda i: (i, 0)),
            pl.BlockSpec(memory_space=pl.ANY),
            pl.BlockSpec((2 * h, c), lambda i: (0, 0)),
            pl.BlockSpec((1, c), lambda i: (0, 0)),
        ],
        out_specs=pl.BlockSpec((bm, c), lambda i: (i, 0)),
        out_shape=jax.ShapeDtypeStruct((n, c), jnp.float32),
        scratch_shapes=[pltpu.VMEM((n, h), jnp.bfloat16),
                        pltpu.SemaphoreType.DMA],
    )(adj_bf, h2bf,
      jnp.concatenate([w12, w22], axis=0).astype(jnp.bfloat16),
      b2.reshape(1, c))
    return out


def kernel(x, adj, emb, W1_1, W2_1, b1, W1_h, W2_h, bh, W1_2, W2_2, b2):
    del x, W1_1, W2_1, b1
    return _forward(adj, emb, W1_h, W2_h, bh, W1_2, W2_2, b2, bm1=200, bm2=400)
